# baseline (speedup 1.0000x reference)
import functools

import jax
import jax.numpy as jnp
from jax import lax
from jax.experimental import pallas as pl
from jax.experimental.pallas import tpu as pltpu
from jax.experimental.pallas import tpu_sc as plsc

V = 25000
C = 100000
E = 800000
F = 64
EPS = 1e-6
ROUNDS = 16

RB_V = 1000
RB_C = 2000
NB_V = V // RB_V
NB_C = C // RB_C

TILES = 16
B = 250
EPT = E // TILES
NBLK = EPT // B
EPT32 = E // 32
NBLK32 = EPT32 // B
ZR = 136
SB = 8
SBD = 4


def _pad128(n):
    return (n + 127) // 128 * 128


def _lrelu(x):
    return jnp.where(x > 0, x, 0.2 * x)


def _softplus(x):
    return jnp.maximum(x, 0.0) + jnp.log1p(jnp.exp(-jnp.abs(x)))


def _sigmoid(x):
    return 1.0 / (1.0 + jnp.exp(-x))


def _sc_mesh():
    return plsc.VectorSubcoreMesh(core_axis_name="c", subcore_axis_name="s",
                                  num_cores=2, num_subcores=TILES)


@functools.lru_cache(maxsize=None)
def _make_seg_sum(n_dst, fc):
    n_pad = _pad128(n_dst)
    rpt = n_pad // TILES
    nbuf = 4 if fc <= 16 else 2
    lead = nbuf // 2
    assert rpt % ZR == 0 and NBLK % SB == 0

    @functools.partial(
        pl.kernel,
        mesh=_sc_mesh(),
        out_type=jax.ShapeDtypeStruct((4 * n_pad, fc), jnp.float32),
        scratch_types=[
            pltpu.VMEM((2, SB, B), jnp.int32),
            pltpu.VMEM((2, SB, B), jnp.int32),
            pltpu.VMEM((nbuf, B, fc), jnp.float32),
            pltpu.VMEM((ZR, fc), jnp.float32),
            pltpu.VMEM_SHARED((n_pad, fc), jnp.float32),
            pltpu.SemaphoreType.DMA,
            pltpu.SemaphoreType.DMA,
            pltpu.SemaphoreType.DMA,
        ],
        compiler_params=pltpu.CompilerParams(use_tc_tiling_on_sc=False, skip_device_barrier=True),
    )
    def k(src_hbm, sidx_hbm, didx_hbm, out_hbm, sidx_v, didx_v, rows_v, zbuf,
          acc, gsem, ssem, isem):
        cid = lax.axis_index("c")
        sid = lax.axis_index("s")
        nsb = NBLK // SB

        def zrow(i, _):
            for h in range(fc // 16):
                zbuf[i, pl.ds(16 * h, 16)] = jnp.zeros((16,), jnp.float32)
            return 0

        lax.fori_loop(0, ZR, zrow, 0)

        for lk in range(2):
            chunk = cid * 2 + lk
            sbase = (chunk * TILES + sid) * NBLK
            dbase = sid * NBLK

            def zcp(r, _):
                pltpu.sync_copy(zbuf, acc.at[pl.ds(sid * rpt + r * ZR, ZR)])
                return 0

            lax.fori_loop(0, rpt // ZR, zcp, 0)
            plsc.subcore_barrier()

            def load_idx(s, p):
                return (pltpu.async_copy(
                            sidx_hbm.at[pl.ds(sbase + s * SB, SB)],
                            sidx_v.at[p], isem),
                        pltpu.async_copy(
                            didx_hbm.at[pl.ds(dbase + s * SB, SB)],
                            didx_v.at[p], isem))

            def pipeline(p):
                gh = [None] * SB
                sh = [None] * SB

                def scat(j):
                    gh[j].wait()
                    sh[j] = pltpu.async_copy(
                        rows_v.at[j % nbuf], acc.at[didx_v.at[p, j]], ssem,
                        add=True)

                for j in range(SB):
                    if j >= nbuf:
                        sh[j - nbuf].wait()
                    gh[j] = pltpu.async_copy(
                        src_hbm.at[sidx_v.at[p, j]], rows_v.at[j % nbuf],
                        gsem)
                    if j >= lead:
                        scat(j - lead)
                for j in range(SB - lead, SB):
                    scat(j)
                for j in range(SB - nbuf, SB):
                    sh[j].wait()

            for h in load_idx(0, 0):
                h.wait()

            def pair(t, _):
                h1 = load_idx(2 * t + 1, 1)
                pipeline(0)
                for h in h1:
                    h.wait()
                h0 = load_idx(2 * t + 2, 0)
                pipeline(1)
                for h in h0:
                    h.wait()
                return 0

            lax.fori_loop(0, (nsb - 1) // 2, pair, 0)
            pipeline(0)
            plsc.subcore_barrier()
            pltpu.sync_copy(
                acc.at[pl.ds(sid * rpt, rpt)],
                out_hbm.at[pl.ds(chunk * n_pad + sid * rpt, rpt)])
            plsc.subcore_barrier()

    return k


@functools.lru_cache(maxsize=None)
def _make_degree():
    n_pad = _pad128(2 * V)
    rpt = n_pad // TILES
    assert rpt % ZR == 0 and NBLK32 % SBD == 0

    @functools.partial(
        pl.kernel,
        mesh=_sc_mesh(),
        out_type=jax.ShapeDtypeStruct((2 * n_pad, 16), jnp.float32),
        scratch_types=[
            pltpu.VMEM((SBD, B), jnp.int32),
            pltpu.VMEM((B, 16), jnp.float32),
            pltpu.VMEM((ZR, 16), jnp.float32),
            pltpu.VMEM_SHARED((n_pad, 16), jnp.float32),
        ],
        compiler_params=pltpu.CompilerParams(use_tc_tiling_on_sc=False, skip_device_barrier=True),
    )
    def k(didx_hbm, out_hbm, didx_v, ones_v, zbuf, acc):
        cid = lax.axis_index("c")
        sid = lax.axis_index("s")
        w = cid * TILES + sid

        def fill(i, _):
            zbuf[i] = jnp.zeros((16,), jnp.float32)
            return 0

        lax.fori_loop(0, ZR, fill, 0)

        def fill1(i, _):
            ones_v[i] = jnp.ones((16,), jnp.float32)
            return 0

        lax.fori_loop(0, B, fill1, 0)

        def zcp(r, _):
            pltpu.sync_copy(zbuf, acc.at[pl.ds(sid * rpt + r * ZR, ZR)])
            return 0

        lax.fori_loop(0, rpt // ZR, zcp, 0)
        plsc.subcore_barrier()

        def sblk(s, _):
            pltpu.sync_copy(didx_hbm.at[pl.ds(w * NBLK32 + s * SBD, SBD)],
                            didx_v)
            for j in range(SBD):
                pltpu.sync_copy(ones_v, acc.at[didx_v.at[j]], add=True)
            return 0

        lax.fori_loop(0, NBLK32 // SBD, sblk, 0)
        plsc.subcore_barrier()
        pltpu.sync_copy(acc.at[pl.ds(sid * rpt, rpt)],
                        out_hbm.at[pl.ds(cid * n_pad + sid * rpt, rpt)])

    return k


def _wb(p):
    return p[0], p[1].reshape(1, -1)


def _pn_fin(x, cs, old, count):
    xx = x - cs * (1.0 / (float(count) * float(count)))
    var = jnp.mean(xx * xx, axis=-1, keepdims=True)
    return xx * lax.rsqrt(var + EPS) * 0.25 + 0.1 * old


def _query_call(variables, noise, wq):
    (w1, b1), (w2, b2) = wq

    def body(v_ref, n_ref, w1r, b1r, w2r, b2r, q_ref, litsp_ref):
        v1 = jnp.concatenate([v_ref[...], n_ref[...]], axis=-1)
        h = _lrelu(jnp.dot(v1, w1r[...], preferred_element_type=jnp.float32)
                   + b1r[...])
        q = jnp.dot(h, w2r[...], preferred_element_type=jnp.float32) + b2r[...]
        q_ref[...] = q
        spq = _softplus(q)
        spn = _softplus(-q)
        for kk in range(4):
            litsp_ref[kk, 0] = spq[:, 16 * kk:16 * (kk + 1)]
            litsp_ref[kk, 1] = spn[:, 16 * kk:16 * (kk + 1)]

    return pl.pallas_call(
        body,
        grid=(NB_V,),
        in_specs=[
            pl.BlockSpec((RB_V, F), lambda i: (i, 0)),
            pl.BlockSpec((RB_V, 4), lambda i: (i, 0)),
            pl.BlockSpec((F + 4, F), lambda i: (0, 0)),
            pl.BlockSpec((1, F), lambda i: (0, 0)),
            pl.BlockSpec((F, F), lambda i: (0, 0)),
            pl.BlockSpec((1, F), lambda i: (0, 0)),
        ],
        out_specs=[
            pl.BlockSpec((RB_V, F), lambda i: (i, 0)),
            pl.BlockSpec((4, 2, RB_V, 16), lambda i: (0, 0, i, 0)),
        ],
        out_shape=[
            jax.ShapeDtypeStruct((V, F), jnp.float32),
            jax.ShapeDtypeStruct((4, 2, V, 16), jnp.float32),
        ],
    )(variables, noise, w1, b1, w2, b2)


def _query_fused_call(nv, csv, var_prev, noise, wq):
    (w1, b1), (w2, b2) = wq

    def body(nv_ref, cs_ref, old_ref, n_ref, w1r, b1r, w2r, b2r,
             v_ref, q_ref, litsp_ref):
        v = _pn_fin(nv_ref[...], cs_ref[...], old_ref[...], V)
        v_ref[...] = v
        v1 = jnp.concatenate([v, n_ref[...]], axis=-1)
        h = _lrelu(jnp.dot(v1, w1r[...], preferred_element_type=jnp.float32)
                   + b1r[...])
        q = jnp.dot(h, w2r[...], preferred_element_type=jnp.float32) + b2r[...]
        q_ref[...] = q
        spq = _softplus(q)
        spn = _softplus(-q)
        for kk in range(4):
            litsp_ref[kk, 0] = spq[:, 16 * kk:16 * (kk + 1)]
            litsp_ref[kk, 1] = spn[:, 16 * kk:16 * (kk + 1)]

    return pl.pallas_call(
        body,
        grid=(NB_V,),
        in_specs=[
            pl.BlockSpec((RB_V, F), lambda i: (i, 0)),
            pl.BlockSpec((1, F), lambda i: (0, 0)),
            pl.BlockSpec((RB_V, F), lambda i: (i, 0)),
            pl.BlockSpec((RB_V, 4), lambda i: (i, 0)),
            pl.BlockSpec((F + 4, F), lambda i: (0, 0)),
            pl.BlockSpec((1, F), lambda i: (0, 0)),
            pl.BlockSpec((F, F), lambda i: (0, 0)),
            pl.BlockSpec((1, F), lambda i: (0, 0)),
        ],
        out_specs=[
            pl.BlockSpec((RB_V, F), lambda i: (i, 0)),
            pl.BlockSpec((RB_V, F), lambda i: (i, 0)),
            pl.BlockSpec((4, 2, RB_V, 16), lambda i: (0, 0, i, 0)),
        ],
        out_shape=[
            jax.ShapeDtypeStruct((V, F), jnp.float32),
            jax.ShapeDtypeStruct((V, F), jnp.float32),
            jax.ShapeDtypeStruct((4, 2, V, 16), jnp.float32),
        ],
    )(nv, csv, var_prev, noise, w1, b1, w2, b2)


def _clause_fused_call(cv4, ncv_p, cs_p, old, wc):
    (w1, b1), (w2, b2) = wc

    def body(cv_ref, np_ref, cp_ref, old_ref, w1r, b1r, w2r, b2r,
             cf_ref, src_ref, ncv_ref, cs_ref):
        cls = _pn_fin(np_ref[...], cp_ref[...], old_ref[...], C)
        cf_ref[...] = cls
        cv = jnp.concatenate([cv_ref[kk] for kk in range(4)], axis=-1)
        closs = jnp.exp(-cv)
        cu = jnp.concatenate([cls, 4.0 * closs], axis=-1)
        h = _lrelu(jnp.dot(cu, w1r[...], preferred_element_type=jnp.float32)
                   + b1r[...])
        cd = jnp.dot(h, w2r[...], preferred_element_type=jnp.float32) + b2r[...]
        vla = cd[:, :F]
        ncv = cd[:, F:]
        ncs = -closs
        src_ref[0] = ncs[:, :32]
        src_ref[1] = ncs[:, 32:]
        src_ref[2] = vla[:, :32]
        src_ref[3] = vla[:, 32:]
        ncv_ref[...] = ncv

        @pl.when(pl.program_id(0) == 0)
        def _():
            cs_ref[...] = jnp.zeros_like(cs_ref)

        cs_ref[...] += jnp.sum(ncv, axis=0, keepdims=True)

    return pl.pallas_call(
        body,
        grid=(NB_C,),
        in_specs=[
            pl.BlockSpec((4, RB_C, 16), lambda i: (0, i, 0)),
            pl.BlockSpec((RB_C, F), lambda i: (i, 0)),
            pl.BlockSpec((1, F), lambda i: (0, 0)),
            pl.BlockSpec((RB_C, F), lambda i: (i, 0)),
            pl.BlockSpec((2 * F, 2 * F), lambda i: (0, 0)),
            pl.BlockSpec((1, 2 * F), lambda i: (0, 0)),
            pl.BlockSpec((2 * F, 2 * F), lambda i: (0, 0)),
            pl.BlockSpec((1, 2 * F), lambda i: (0, 0)),
        ],
        out_specs=[
            pl.BlockSpec((RB_C, F), lambda i: (i, 0)),
            pl.BlockSpec((4, RB_C, 32), lambda i: (0, i, 0)),
            pl.BlockSpec((RB_C, F), lambda i: (i, 0)),
            pl.BlockSpec((1, F), lambda i: (0, 0)),
        ],
        out_shape=[
            jax.ShapeDtypeStruct((C, F), jnp.float32),
            jax.ShapeDtypeStruct((4, C, 32), jnp.float32),
            jax.ShapeDtypeStruct((C, F), jnp.float32),
            jax.ShapeDtypeStruct((1, F), jnp.float32),
        ],
    )(cv4, ncv_p, cs_p, old, w1, b1, w2, b2)


def _out_fused_call(ncv_p, cs_p, old, wo):
    (w1, b1), (w2, b2) = wo

    def body(np_ref, cp_ref, old_ref, w1r, b1r, w2r, b2r, sig_ref, sp_ref):
        cls = _pn_fin(np_ref[...], cp_ref[...], old_ref[...], C)
        h = _lrelu(jnp.dot(cls, w1r[...],
                           preferred_element_type=jnp.float32) + b1r[...])
        logit = jnp.dot(h, w2r[...], preferred_element_type=jnp.float32) + b2r[...]
        sig_ref[...] = _sigmoid(logit)
        sp_ref[...] = _softplus(logit)

    return pl.pallas_call(
        body,
        grid=(NB_C,),
        in_specs=[
            pl.BlockSpec((RB_C, F), lambda i: (i, 0)),
            pl.BlockSpec((1, F), lambda i: (0, 0)),
            pl.BlockSpec((RB_C, F), lambda i: (i, 0)),
            pl.BlockSpec((F, F), lambda i: (0, 0)),
            pl.BlockSpec((1, F), lambda i: (0, 0)),
            pl.BlockSpec((F, 1), lambda i: (0, 0)),
            pl.BlockSpec((1, 1), lambda i: (0, 0)),
        ],
        out_specs=[
            pl.BlockSpec((RB_C, 1), lambda i: (i, 0)),
            pl.BlockSpec((RB_C, 1), lambda i: (i, 0)),
        ],
        out_shape=[
            jax.ShapeDtypeStruct((C, 1), jnp.float32),
            jax.ShapeDtypeStruct((C, 1), jnp.float32),
        ],
    )(ncv_p, cs_p, old, w1, b1, w2, b2)


def _clause_call(cv4, clauses, wc):
    (w1, b1), (w2, b2) = wc

    def body(cv_ref, c_ref, w1r, b1r, w2r, b2r, src_ref, ncv_ref, cs_ref):
        cv = jnp.concatenate([cv_ref[kk] for kk in range(4)], axis=-1)
        closs = jnp.exp(-cv)
        cu = jnp.concatenate([c_ref[...], 4.0 * closs], axis=-1)
        h = _lrelu(jnp.dot(cu, w1r[...], preferred_element_type=jnp.float32)
                   + b1r[...])
        cd = jnp.dot(h, w2r[...], preferred_element_type=jnp.float32) + b2r[...]
        vla = cd[:, :F]
        ncv = cd[:, F:]
        ncs = -closs
        src_ref[0] = ncs[:, :32]
        src_ref[1] = ncs[:, 32:]
        src_ref[2] = vla[:, :32]
        src_ref[3] = vla[:, 32:]
        ncv_ref[...] = ncv

        @pl.when(pl.program_id(0) == 0)
        def _():
            cs_ref[...] = jnp.zeros_like(cs_ref)

        cs_ref[...] += jnp.sum(ncv, axis=0, keepdims=True)

    return pl.pallas_call(
        body,
        grid=(NB_C,),
        in_specs=[
            pl.BlockSpec((4, RB_C, 16), lambda i: (0, i, 0)),
            pl.BlockSpec((RB_C, F), lambda i: (i, 0)),
            pl.BlockSpec((2 * F, 2 * F), lambda i: (0, 0)),
            pl.BlockSpec((1, 2 * F), lambda i: (0, 0)),
            pl.BlockSpec((2 * F, 2 * F), lambda i: (0, 0)),
            pl.BlockSpec((1, 2 * F), lambda i: (0, 0)),
        ],
        out_specs=[
            pl.BlockSpec((4, RB_C, 32), lambda i: (0, i, 0)),
            pl.BlockSpec((RB_C, F), lambda i: (i, 0)),
            pl.BlockSpec((1, F), lambda i: (0, 0)),
        ],
        out_shape=[
            jax.ShapeDtypeStruct((4, C, 32), jnp.float32),
            jax.ShapeDtypeStruct((C, F), jnp.float32),
            jax.ShapeDtypeStruct((1, F), jnp.float32),
        ],
    )(cv4, clauses, w1, b1, w2, b2)


def _update_call(q, acc4, variables, deg4, wg):
    (w1, b1), (w2, b2), (w3, b3) = wg

    def body(q_ref, a_ref, v_ref, d_ref, w1r, b1r, w2r, b2r, w3r, b3r,
             nv_ref, cs_ref):
        q = q_ref[...]
        t_pos = jnp.concatenate([a_ref[0, 0], a_ref[1, 0]], axis=-1)
        t_neg = jnp.concatenate([a_ref[0, 1], a_ref[1, 1]], axis=-1)
        vlp = jnp.concatenate([a_ref[2, 0], a_ref[3, 0]], axis=-1)
        vln = jnp.concatenate([a_ref[2, 1], a_ref[3, 1]], axis=-1)
        deg = d_ref[0] + d_ref[1]
        degp = deg[0][:, 0:1]
        degn = deg[1][:, 0:1]
        dwp = lax.rsqrt(jnp.maximum(degp, 1.0))
        dwn = lax.rsqrt(jnp.maximum(degn, 1.0))
        vdw = 4.0 * lax.rsqrt(jnp.maximum(degp + degn, 1.0))
        qg = _sigmoid(q) * t_pos - _sigmoid(-q) * t_neg
        unit = jnp.concatenate(
            [qg * vdw, v_ref[...], vlp * dwp, vln * dwn], axis=-1)
        h1 = _lrelu(jnp.dot(unit, w1r[...], preferred_element_type=jnp.float32)
                    + b1r[...])
        h2 = _lrelu(jnp.dot(h1, w2r[...], preferred_element_type=jnp.float32)
                    + b2r[...])
        nv = jnp.dot(h2, w3r[...], preferred_element_type=jnp.float32) + b3r[...]
        nv_ref[...] = nv

        @pl.when(pl.program_id(0) == 0)
        def _():
            cs_ref[...] = jnp.zeros_like(cs_ref)

        cs_ref[...] += jnp.sum(nv, axis=0, keepdims=True)

    return pl.pallas_call(
        body,
        grid=(NB_V,),
        in_specs=[
            pl.BlockSpec((RB_V, F), lambda i: (i, 0)),
            pl.BlockSpec((4, 2, RB_V, 32), lambda i: (0, 0, i, 0)),
            pl.BlockSpec((RB_V, F), lambda i: (i, 0)),
            pl.BlockSpec((2, 2, RB_V, 16), lambda i: (0, 0, i, 0)),
            pl.BlockSpec((4 * F, 2 * F), lambda i: (0, 0)),
            pl.BlockSpec((1, 2 * F), lambda i: (0, 0)),
            pl.BlockSpec((2 * F, 2 * F), lambda i: (0, 0)),
            pl.BlockSpec((1, 2 * F), lambda i: (0, 0)),
            pl.BlockSpec((2 * F, F), lambda i: (0, 0)),
            pl.BlockSpec((1, F), lambda i: (0, 0)),
        ],
        out_specs=[
            pl.BlockSpec((RB_V, F), lambda i: (i, 0)),
            pl.BlockSpec((1, F), lambda i: (0, 0)),
        ],
        out_shape=[
            jax.ShapeDtypeStruct((V, F), jnp.float32),
            jax.ShapeDtypeStruct((1, F), jnp.float32),
        ],
    )(q, acc4, variables, deg4, w1, b1, w2, b2, w3, b3)


def kernel(params, lit_idx, clause_idx, clause_graph_ids, var_graph_ids):
    wq = tuple(_wb(p) for p in params['variables_query'])
    wc = tuple(_wb(p) for p in params['clause_mlp'])
    wg = tuple(_wb(p) for p in params['update_gate'])
    wo = tuple(_wb(p) for p in params['clauses_output'])

    noise = jnp.stack([
        jax.random.normal(jax.random.fold_in(jax.random.key(42), s), (V, 4),
                          dtype=jnp.float32) for s in range(ROUNDS)])

    off_a = (jnp.arange(4, dtype=jnp.int32) * (2 * V))[:, None]
    sidx_a = (lit_idx[None, :] + off_a).reshape(4 * TILES * NBLK, B)
    didx_a = clause_idx.reshape(TILES * NBLK, B)
    off_b = (jnp.arange(4, dtype=jnp.int32) * C)[:, None]
    sidx_b = (clause_idx[None, :] + off_b).reshape(4 * TILES * NBLK, B)
    didx_b = lit_idx.reshape(TILES * NBLK, B)
    didx_d = lit_idx.reshape(32 * NBLK32, B)

    pad_l = _pad128(2 * V)
    pad_c = _pad128(C)
    deg4 = (_make_degree()(didx_d)
            .reshape(2, pad_l, 16)[:, :2 * V]
            .reshape(2, 2, V, 16))

    seg_a = _make_seg_sum(C, 16)
    seg_b = _make_seg_sum(2 * V, 32)

    ones_v = jnp.ones((V, F), jnp.float32)
    ones_c = jnp.ones((C, F), jnp.float32)

    variables = ones_v
    var_old = ones_v
    cls_old = ones_c
    nv = cs_v = ncv = cs_c = None

    for step in range(ROUNDS):
        if step == 0:
            q, litsp = _query_call(variables, noise[step], wq)
        else:
            variables, q, litsp = _query_fused_call(
                nv, cs_v, var_old, noise[step], wq)
        cv4 = (seg_a(litsp.reshape(8 * V, 16), sidx_a, didx_a)
               .reshape(4, pad_c, 16)[:, :C])
        if step == 0:
            src4, ncv, cs_c = _clause_call(cv4, cls_old, wc)
        else:
            cls_old, src4, ncv, cs_c = _clause_fused_call(
                cv4, ncv, cs_c, cls_old, wc)
        acc4 = (seg_b(src4.reshape(4 * C, 32), sidx_b, didx_b)
                .reshape(4, pad_l, 32)[:, :2 * V]
                .reshape(4, 2, V, 32))
        var_old = variables
        nv, cs_v = _update_call(q, acc4, variables, deg4, wg)

    sig, sp = _out_fused_call(ncv, cs_c, cls_old, wo)
    return (sig[:, 0], sp[:, 0])


# speedup vs baseline: 1.0951x; 1.0951x over previous
import functools

import jax
import jax.numpy as jnp
from jax import lax
from jax.experimental import pallas as pl
from jax.experimental.pallas import tpu as pltpu
from jax.experimental.pallas import tpu_sc as plsc

V = 25000
C = 100000
E = 800000
F = 64
EPS = 1e-6
ROUNDS = 16

RB_V = 1000
RB_C = 2000
NB_V = V // RB_V
NB_C = C // RB_C

TILES = 16
B = 250
EPT = E // TILES
NBLK = EPT // B
EPT32 = E // 32
NBLK32 = EPT32 // B
ZR = 136
SB = 8
SBD = 4


def _pad128(n):
    return (n + 127) // 128 * 128


def _lrelu(x):
    return jnp.where(x > 0, x, 0.2 * x)


def _softplus(x):
    return jnp.maximum(x, 0.0) + jnp.log1p(jnp.exp(-jnp.abs(x)))


def _sigmoid(x):
    return 1.0 / (1.0 + jnp.exp(-x))


def _sc_mesh():
    return plsc.VectorSubcoreMesh(core_axis_name="c", subcore_axis_name="s",
                                  num_cores=2, num_subcores=TILES)


@functools.lru_cache(maxsize=None)
def _make_seg_sum(n_dst, fc):
    n_pad = _pad128(n_dst)
    rpt = n_pad // TILES
    nbuf = 4 if fc <= 16 else 2
    lead = nbuf // 2
    assert rpt % ZR == 0 and NBLK % SB == 0

    @functools.partial(
        pl.kernel,
        mesh=_sc_mesh(),
        out_type=jax.ShapeDtypeStruct((4 * n_pad, fc), jnp.float32),
        scratch_types=[
            pltpu.VMEM((2, SB, B), jnp.int32),
            pltpu.VMEM((2, SB, B), jnp.int32),
            pltpu.VMEM((nbuf, B, fc), jnp.float32),
            pltpu.VMEM((ZR, fc), jnp.float32),
            pltpu.VMEM_SHARED((n_pad, fc), jnp.float32),
            pltpu.SemaphoreType.DMA,
            pltpu.SemaphoreType.DMA,
            pltpu.SemaphoreType.DMA,
        ],
        compiler_params=pltpu.CompilerParams(use_tc_tiling_on_sc=False),
    )
    def k(src_hbm, sidx_hbm, didx_hbm, out_hbm, sidx_v, didx_v, rows_v, zbuf,
          acc, gsem, ssem, isem):
        cid = lax.axis_index("c")
        sid = lax.axis_index("s")
        nsb = NBLK // SB

        def zrow(i, _):
            for h in range(fc // 16):
                zbuf[i, pl.ds(16 * h, 16)] = jnp.zeros((16,), jnp.float32)
            return 0

        lax.fori_loop(0, ZR, zrow, 0)

        for lk in range(2):
            chunk = cid * 2 + lk
            sbase = (chunk * TILES + sid) * NBLK
            dbase = sid * NBLK

            def zcp(r, _):
                pltpu.sync_copy(zbuf, acc.at[pl.ds(sid * rpt + r * ZR, ZR)])
                return 0

            lax.fori_loop(0, rpt // ZR, zcp, 0)
            plsc.subcore_barrier()

            def load_idx(s, p):
                return (pltpu.async_copy(
                            sidx_hbm.at[pl.ds(sbase + s * SB, SB)],
                            sidx_v.at[p], isem),
                        pltpu.async_copy(
                            didx_hbm.at[pl.ds(dbase + s * SB, SB)],
                            didx_v.at[p], isem))

            def pipeline(p):
                gh = [None] * SB
                sh = [None] * SB

                def scat(j):
                    gh[j].wait()
                    sh[j] = pltpu.async_copy(
                        rows_v.at[j % nbuf], acc.at[didx_v.at[p, j]], ssem,
                        add=True)

                for j in range(SB):
                    if j >= nbuf:
                        sh[j - nbuf].wait()
                    gh[j] = pltpu.async_copy(
                        src_hbm.at[sidx_v.at[p, j]], rows_v.at[j % nbuf],
                        gsem)
                    if j >= lead:
                        scat(j - lead)
                for j in range(SB - lead, SB):
                    scat(j)
                for j in range(SB - nbuf, SB):
                    sh[j].wait()

            for h in load_idx(0, 0):
                h.wait()

            def pair(t, _):
                h1 = load_idx(2 * t + 1, 1)
                pipeline(0)
                for h in h1:
                    h.wait()
                h0 = load_idx(2 * t + 2, 0)
                pipeline(1)
                for h in h0:
                    h.wait()
                return 0

            lax.fori_loop(0, (nsb - 1) // 2, pair, 0)
            pipeline(0)
            plsc.subcore_barrier()
            pltpu.sync_copy(
                acc.at[pl.ds(sid * rpt, rpt)],
                out_hbm.at[pl.ds(chunk * n_pad + sid * rpt, rpt)])
            plsc.subcore_barrier()

    return k


@functools.lru_cache(maxsize=None)
def _make_degree():
    n_pad = _pad128(2 * V)
    rpt = n_pad // TILES
    assert rpt % ZR == 0 and NBLK32 % SBD == 0

    @functools.partial(
        pl.kernel,
        mesh=_sc_mesh(),
        out_type=jax.ShapeDtypeStruct((2 * n_pad, 16), jnp.float32),
        scratch_types=[
            pltpu.VMEM((SBD, B), jnp.int32),
            pltpu.VMEM((B, 16), jnp.float32),
            pltpu.VMEM((ZR, 16), jnp.float32),
            pltpu.VMEM_SHARED((n_pad, 16), jnp.float32),
        ],
        compiler_params=pltpu.CompilerParams(use_tc_tiling_on_sc=False),
    )
    def k(didx_hbm, out_hbm, didx_v, ones_v, zbuf, acc):
        cid = lax.axis_index("c")
        sid = lax.axis_index("s")
        w = cid * TILES + sid

        def fill(i, _):
            zbuf[i] = jnp.zeros((16,), jnp.float32)
            return 0

        lax.fori_loop(0, ZR, fill, 0)

        def fill1(i, _):
            ones_v[i] = jnp.ones((16,), jnp.float32)
            return 0

        lax.fori_loop(0, B, fill1, 0)

        def zcp(r, _):
            pltpu.sync_copy(zbuf, acc.at[pl.ds(sid * rpt + r * ZR, ZR)])
            return 0

        lax.fori_loop(0, rpt // ZR, zcp, 0)
        plsc.subcore_barrier()

        def sblk(s, _):
            pltpu.sync_copy(didx_hbm.at[pl.ds(w * NBLK32 + s * SBD, SBD)],
                            didx_v)
            for j in range(SBD):
                pltpu.sync_copy(ones_v, acc.at[didx_v.at[j]], add=True)
            return 0

        lax.fori_loop(0, NBLK32 // SBD, sblk, 0)
        plsc.subcore_barrier()
        pltpu.sync_copy(acc.at[pl.ds(sid * rpt, rpt)],
                        out_hbm.at[pl.ds(cid * n_pad + sid * rpt, rpt)])

    return k


def _wb(p):
    return p[0], p[1].reshape(1, -1)


def _pn_fin(x, cs, old, count):
    xx = x - cs * (1.0 / (float(count) * float(count)))
    var = jnp.mean(xx * xx, axis=-1, keepdims=True)
    return xx * lax.rsqrt(var + EPS) * 0.25 + 0.1 * old


def _query_call(variables, noise, wq):
    (w1, b1), (w2, b2) = wq

    def body(v_ref, n_ref, w1r, b1r, w2r, b2r, q_ref, litsp_ref):
        v1 = jnp.concatenate([v_ref[...], n_ref[...]], axis=-1)
        h = _lrelu(jnp.dot(v1, w1r[...], preferred_element_type=jnp.float32)
                   + b1r[...])
        q = jnp.dot(h, w2r[...], preferred_element_type=jnp.float32) + b2r[...]
        q_ref[...] = q
        spq = _softplus(q)
        spn = _softplus(-q)
        for kk in range(4):
            litsp_ref[kk, 0] = spq[:, 16 * kk:16 * (kk + 1)]
            litsp_ref[kk, 1] = spn[:, 16 * kk:16 * (kk + 1)]

    return pl.pallas_call(
        body,
        grid=(NB_V,),
        in_specs=[
            pl.BlockSpec((RB_V, F), lambda i: (i, 0)),
            pl.BlockSpec((RB_V, 4), lambda i: (i, 0)),
            pl.BlockSpec((F + 4, F), lambda i: (0, 0)),
            pl.BlockSpec((1, F), lambda i: (0, 0)),
            pl.BlockSpec((F, F), lambda i: (0, 0)),
            pl.BlockSpec((1, F), lambda i: (0, 0)),
        ],
        out_specs=[
            pl.BlockSpec((RB_V, F), lambda i: (i, 0)),
            pl.BlockSpec((4, 2, RB_V, 16), lambda i: (0, 0, i, 0)),
        ],
        out_shape=[
            jax.ShapeDtypeStruct((V, F), jnp.float32),
            jax.ShapeDtypeStruct((4, 2, V, 16), jnp.float32),
        ],
    )(variables, noise, w1, b1, w2, b2)


def _query_fused_call(nv, csv, var_prev, noise, wq):
    (w1, b1), (w2, b2) = wq

    def body(nv_ref, cs_ref, old_ref, n_ref, w1r, b1r, w2r, b2r,
             v_ref, q_ref, litsp_ref):
        v = _pn_fin(nv_ref[...], cs_ref[...], old_ref[...], V)
        v_ref[...] = v
        v1 = jnp.concatenate([v, n_ref[...]], axis=-1)
        h = _lrelu(jnp.dot(v1, w1r[...], preferred_element_type=jnp.float32)
                   + b1r[...])
        q = jnp.dot(h, w2r[...], preferred_element_type=jnp.float32) + b2r[...]
        q_ref[...] = q
        spq = _softplus(q)
        spn = _softplus(-q)
        for kk in range(4):
            litsp_ref[kk, 0] = spq[:, 16 * kk:16 * (kk + 1)]
            litsp_ref[kk, 1] = spn[:, 16 * kk:16 * (kk + 1)]

    return pl.pallas_call(
        body,
        grid=(NB_V,),
        in_specs=[
            pl.BlockSpec((RB_V, F), lambda i: (i, 0)),
            pl.BlockSpec((1, F), lambda i: (0, 0)),
            pl.BlockSpec((RB_V, F), lambda i: (i, 0)),
            pl.BlockSpec((RB_V, 4), lambda i: (i, 0)),
            pl.BlockSpec((F + 4, F), lambda i: (0, 0)),
            pl.BlockSpec((1, F), lambda i: (0, 0)),
            pl.BlockSpec((F, F), lambda i: (0, 0)),
            pl.BlockSpec((1, F), lambda i: (0, 0)),
        ],
        out_specs=[
            pl.BlockSpec((RB_V, F), lambda i: (i, 0)),
            pl.BlockSpec((RB_V, F), lambda i: (i, 0)),
            pl.BlockSpec((4, 2, RB_V, 16), lambda i: (0, 0, i, 0)),
        ],
        out_shape=[
            jax.ShapeDtypeStruct((V, F), jnp.float32),
            jax.ShapeDtypeStruct((V, F), jnp.float32),
            jax.ShapeDtypeStruct((4, 2, V, 16), jnp.float32),
        ],
    )(nv, csv, var_prev, noise, w1, b1, w2, b2)


def _clause_fused_call(cv4, ncv_p, cs_p, old, wc):
    (w1, b1), (w2, b2) = wc

    def body(cv_ref, np_ref, cp_ref, old_ref, w1r, b1r, w2r, b2r,
             cf_ref, src_ref, ncv_ref, cs_ref):
        cls = _pn_fin(np_ref[...], cp_ref[...], old_ref[...], C)
        cf_ref[...] = cls
        cv = jnp.concatenate([cv_ref[kk] for kk in range(4)], axis=-1)
        closs = jnp.exp(-cv)
        cu = jnp.concatenate([cls, 4.0 * closs], axis=-1)
        h = _lrelu(jnp.dot(cu, w1r[...], preferred_element_type=jnp.float32)
                   + b1r[...])
        cd = jnp.dot(h, w2r[...], preferred_element_type=jnp.float32) + b2r[...]
        vla = cd[:, :F]
        ncv = cd[:, F:]
        ncs = -closs
        src_ref[0] = ncs[:, :32]
        src_ref[1] = ncs[:, 32:]
        src_ref[2] = vla[:, :32]
        src_ref[3] = vla[:, 32:]
        ncv_ref[...] = ncv

        @pl.when(pl.program_id(0) == 0)
        def _():
            cs_ref[...] = jnp.zeros_like(cs_ref)

        cs_ref[...] += jnp.sum(ncv, axis=0, keepdims=True)

    return pl.pallas_call(
        body,
        grid=(NB_C,),
        in_specs=[
            pl.BlockSpec((4, RB_C, 16), lambda i: (0, i, 0)),
            pl.BlockSpec((RB_C, F), lambda i: (i, 0)),
            pl.BlockSpec((1, F), lambda i: (0, 0)),
            pl.BlockSpec((RB_C, F), lambda i: (i, 0)),
            pl.BlockSpec((2 * F, 2 * F), lambda i: (0, 0)),
            pl.BlockSpec((1, 2 * F), lambda i: (0, 0)),
            pl.BlockSpec((2 * F, 2 * F), lambda i: (0, 0)),
            pl.BlockSpec((1, 2 * F), lambda i: (0, 0)),
        ],
        out_specs=[
            pl.BlockSpec((RB_C, F), lambda i: (i, 0)),
            pl.BlockSpec((4, RB_C, 32), lambda i: (0, i, 0)),
            pl.BlockSpec((RB_C, F), lambda i: (i, 0)),
            pl.BlockSpec((1, F), lambda i: (0, 0)),
        ],
        out_shape=[
            jax.ShapeDtypeStruct((C, F), jnp.float32),
            jax.ShapeDtypeStruct((4, C, 32), jnp.float32),
            jax.ShapeDtypeStruct((C, F), jnp.float32),
            jax.ShapeDtypeStruct((1, F), jnp.float32),
        ],
    )(cv4, ncv_p, cs_p, old, w1, b1, w2, b2)


def _out_fused_call(ncv_p, cs_p, old, wo):
    (w1, b1), (w2, b2) = wo

    def body(np_ref, cp_ref, old_ref, w1r, b1r, w2r, b2r, sig_ref, sp_ref):
        cls = _pn_fin(np_ref[...], cp_ref[...], old_ref[...], C)
        h = _lrelu(jnp.dot(cls, w1r[...],
                           preferred_element_type=jnp.float32) + b1r[...])
        logit = jnp.dot(h, w2r[...], preferred_element_type=jnp.float32) + b2r[...]
        sig_ref[...] = _sigmoid(logit)
        sp_ref[...] = _softplus(logit)

    return pl.pallas_call(
        body,
        grid=(NB_C,),
        in_specs=[
            pl.BlockSpec((RB_C, F), lambda i: (i, 0)),
            pl.BlockSpec((1, F), lambda i: (0, 0)),
            pl.BlockSpec((RB_C, F), lambda i: (i, 0)),
            pl.BlockSpec((F, F), lambda i: (0, 0)),
            pl.BlockSpec((1, F), lambda i: (0, 0)),
            pl.BlockSpec((F, 1), lambda i: (0, 0)),
            pl.BlockSpec((1, 1), lambda i: (0, 0)),
        ],
        out_specs=[
            pl.BlockSpec((RB_C, 1), lambda i: (i, 0)),
            pl.BlockSpec((RB_C, 1), lambda i: (i, 0)),
        ],
        out_shape=[
            jax.ShapeDtypeStruct((C, 1), jnp.float32),
            jax.ShapeDtypeStruct((C, 1), jnp.float32),
        ],
    )(ncv_p, cs_p, old, w1, b1, w2, b2)


def _clause_call(cv4, clauses, wc):
    (w1, b1), (w2, b2) = wc

    def body(cv_ref, c_ref, w1r, b1r, w2r, b2r, src_ref, ncv_ref, cs_ref):
        cv = jnp.concatenate([cv_ref[kk] for kk in range(4)], axis=-1)
        closs = jnp.exp(-cv)
        cu = jnp.concatenate([c_ref[...], 4.0 * closs], axis=-1)
        h = _lrelu(jnp.dot(cu, w1r[...], preferred_element_type=jnp.float32)
                   + b1r[...])
        cd = jnp.dot(h, w2r[...], preferred_element_type=jnp.float32) + b2r[...]
        vla = cd[:, :F]
        ncv = cd[:, F:]
        ncs = -closs
        src_ref[0] = ncs[:, :32]
        src_ref[1] = ncs[:, 32:]
        src_ref[2] = vla[:, :32]
        src_ref[3] = vla[:, 32:]
        ncv_ref[...] = ncv

        @pl.when(pl.program_id(0) == 0)
        def _():
            cs_ref[...] = jnp.zeros_like(cs_ref)

        cs_ref[...] += jnp.sum(ncv, axis=0, keepdims=True)

    return pl.pallas_call(
        body,
        grid=(NB_C,),
        in_specs=[
            pl.BlockSpec((4, RB_C, 16), lambda i: (0, i, 0)),
            pl.BlockSpec((RB_C, F), lambda i: (i, 0)),
            pl.BlockSpec((2 * F, 2 * F), lambda i: (0, 0)),
            pl.BlockSpec((1, 2 * F), lambda i: (0, 0)),
            pl.BlockSpec((2 * F, 2 * F), lambda i: (0, 0)),
            pl.BlockSpec((1, 2 * F), lambda i: (0, 0)),
        ],
        out_specs=[
            pl.BlockSpec((4, RB_C, 32), lambda i: (0, i, 0)),
            pl.BlockSpec((RB_C, F), lambda i: (i, 0)),
            pl.BlockSpec((1, F), lambda i: (0, 0)),
        ],
        out_shape=[
            jax.ShapeDtypeStruct((4, C, 32), jnp.float32),
            jax.ShapeDtypeStruct((C, F), jnp.float32),
            jax.ShapeDtypeStruct((1, F), jnp.float32),
        ],
    )(cv4, clauses, w1, b1, w2, b2)


def _update_call(q, acc4, variables, deg4, wg):
    (w1, b1), (w2, b2), (w3, b3) = wg

    def body(q_ref, a_ref, v_ref, d_ref, w1r, b1r, w2r, b2r, w3r, b3r,
             nv_ref, cs_ref):
        q = q_ref[...]
        t_pos = jnp.concatenate([a_ref[0, 0], a_ref[1, 0]], axis=-1)
        t_neg = jnp.concatenate([a_ref[0, 1], a_ref[1, 1]], axis=-1)
        vlp = jnp.concatenate([a_ref[2, 0], a_ref[3, 0]], axis=-1)
        vln = jnp.concatenate([a_ref[2, 1], a_ref[3, 1]], axis=-1)
        deg = d_ref[0] + d_ref[1]
        degp = deg[0][:, 0:1]
        degn = deg[1][:, 0:1]
        dwp = lax.rsqrt(jnp.maximum(degp, 1.0))
        dwn = lax.rsqrt(jnp.maximum(degn, 1.0))
        vdw = 4.0 * lax.rsqrt(jnp.maximum(degp + degn, 1.0))
        qg = _sigmoid(q) * t_pos - _sigmoid(-q) * t_neg
        unit = jnp.concatenate(
            [qg * vdw, v_ref[...], vlp * dwp, vln * dwn], axis=-1)
        h1 = _lrelu(jnp.dot(unit, w1r[...], preferred_element_type=jnp.float32)
                    + b1r[...])
        h2 = _lrelu(jnp.dot(h1, w2r[...], preferred_element_type=jnp.float32)
                    + b2r[...])
        nv = jnp.dot(h2, w3r[...], preferred_element_type=jnp.float32) + b3r[...]
        nv_ref[...] = nv

        @pl.when(pl.program_id(0) == 0)
        def _():
            cs_ref[...] = jnp.zeros_like(cs_ref)

        cs_ref[...] += jnp.sum(nv, axis=0, keepdims=True)

    return pl.pallas_call(
        body,
        grid=(NB_V,),
        in_specs=[
            pl.BlockSpec((RB_V, F), lambda i: (i, 0)),
            pl.BlockSpec((4, 2, RB_V, 32), lambda i: (0, 0, i, 0)),
            pl.BlockSpec((RB_V, F), lambda i: (i, 0)),
            pl.BlockSpec((2, 2, RB_V, 16), lambda i: (0, 0, i, 0)),
            pl.BlockSpec((4 * F, 2 * F), lambda i: (0, 0)),
            pl.BlockSpec((1, 2 * F), lambda i: (0, 0)),
            pl.BlockSpec((2 * F, 2 * F), lambda i: (0, 0)),
            pl.BlockSpec((1, 2 * F), lambda i: (0, 0)),
            pl.BlockSpec((2 * F, F), lambda i: (0, 0)),
            pl.BlockSpec((1, F), lambda i: (0, 0)),
        ],
        out_specs=[
            pl.BlockSpec((RB_V, F), lambda i: (i, 0)),
            pl.BlockSpec((1, F), lambda i: (0, 0)),
        ],
        out_shape=[
            jax.ShapeDtypeStruct((V, F), jnp.float32),
            jax.ShapeDtypeStruct((1, F), jnp.float32),
        ],
    )(q, acc4, variables, deg4, w1, b1, w2, b2, w3, b3)


def kernel(params, lit_idx, clause_idx, clause_graph_ids, var_graph_ids):
    wq = tuple(_wb(p) for p in params['variables_query'])
    wc = tuple(_wb(p) for p in params['clause_mlp'])
    wg = tuple(_wb(p) for p in params['update_gate'])
    wo = tuple(_wb(p) for p in params['clauses_output'])

    noise = jnp.stack([
        jax.random.normal(jax.random.fold_in(jax.random.key(42), s), (V, 4),
                          dtype=jnp.float32) for s in range(ROUNDS)])

    off_a = (jnp.arange(4, dtype=jnp.int32) * (2 * V))[:, None]
    sidx_a = (lit_idx[None, :] + off_a).reshape(4 * TILES * NBLK, B)
    didx_a = clause_idx.reshape(TILES * NBLK, B)
    off_b = (jnp.arange(4, dtype=jnp.int32) * C)[:, None]
    sidx_b = (clause_idx[None, :] + off_b).reshape(4 * TILES * NBLK, B)
    didx_b = lit_idx.reshape(TILES * NBLK, B)
    didx_d = lit_idx.reshape(32 * NBLK32, B)

    pad_l = _pad128(2 * V)
    pad_c = _pad128(C)
    deg4 = (_make_degree()(didx_d)
            .reshape(2, pad_l, 16)[:, :2 * V]
            .reshape(2, 2, V, 16))

    seg_a = _make_seg_sum(C, 16)
    seg_b = _make_seg_sum(2 * V, 32)

    ones_v = jnp.ones((V, F), jnp.float32)
    ones_c = jnp.ones((C, F), jnp.float32)

    variables = ones_v
    var_old = ones_v
    cls_old = ones_c
    nv = cs_v = ncv = cs_c = None

    for step in range(ROUNDS):
        if step == 0:
            q, litsp = _query_call(variables, noise[step], wq)
        else:
            variables, q, litsp = _query_fused_call(
                nv, cs_v, var_old, noise[step], wq)
        cv4 = seg_a(litsp.reshape(8 * V, 16), sidx_a, didx_a).reshape(
            4, pad_c, 16)
        if step == 0:
            src4, ncv, cs_c = _clause_call(cv4, cls_old, wc)
        else:
            cls_old, src4, ncv, cs_c = _clause_fused_call(
                cv4, ncv, cs_c, cls_old, wc)
        acc4 = (seg_b(src4.reshape(4 * C, 32), sidx_b, didx_b)
                .reshape(4, pad_l, 32)[:, :2 * V]
                .reshape(4, 2, V, 32))
        var_old = variables
        nv, cs_v = _update_call(q, acc4, variables, deg4, wg)

    sig, sp = _out_fused_call(ncv, cs_c, cls_old, wo)
    return (sig[:, 0], sp[:, 0])


# speedup vs baseline: 1.1463x; 1.0467x over previous
import functools

import jax
import jax.numpy as jnp
from jax import lax
from jax.experimental import pallas as pl
from jax.experimental.pallas import tpu as pltpu
from jax.experimental.pallas import tpu_sc as plsc

V = 25000
C = 100000
E = 800000
F = 64
EPS = 1e-6
ROUNDS = 16

RB_V = 1000
RB_C = 2000
NB_V = V // RB_V
NB_C = C // RB_C

TILES = 16
B = 250
EPT = E // TILES
NBLK = EPT // B
EPT32 = E // 32
NBLK32 = EPT32 // B
ZR = 136
SB = 8
SBD = 4


def _pad128(n):
    return (n + 127) // 128 * 128


def _lrelu(x):
    return jnp.where(x > 0, x, 0.2 * x)


def _softplus(x):
    return jnp.maximum(x, 0.0) + jnp.log1p(jnp.exp(-jnp.abs(x)))


def _sigmoid(x):
    return 1.0 / (1.0 + jnp.exp(-x))


def _sc_mesh():
    return plsc.VectorSubcoreMesh(core_axis_name="c", subcore_axis_name="s",
                                  num_cores=2, num_subcores=TILES)


@functools.lru_cache(maxsize=None)
def _make_seg_sum(n_dst, fc):
    n_pad = _pad128(n_dst)
    rpt = n_pad // TILES
    nbuf = 4 if fc <= 16 else 2
    lead = nbuf // 2
    assert rpt % ZR == 0 and NBLK % SB == 0

    @functools.partial(
        pl.kernel,
        mesh=_sc_mesh(),
        out_type=jax.ShapeDtypeStruct((4 * n_pad, fc), jnp.float32),
        scratch_types=[
            pltpu.VMEM((2, SB, B), jnp.int32),
            pltpu.VMEM((2, SB, B), jnp.int32),
            pltpu.VMEM((nbuf, B, fc), jnp.float32),
            pltpu.VMEM((ZR, fc), jnp.float32),
            pltpu.VMEM_SHARED((n_pad, fc), jnp.float32),
            pltpu.SemaphoreType.DMA,
            pltpu.SemaphoreType.DMA,
            pltpu.SemaphoreType.DMA,
        ],
        compiler_params=pltpu.CompilerParams(use_tc_tiling_on_sc=False),
    )
    def k(src_hbm, sidx_hbm, didx_hbm, out_hbm, sidx_v, didx_v, rows_v, zbuf,
          acc, gsem, ssem, isem):
        cid = lax.axis_index("c")
        sid = lax.axis_index("s")
        nsb = NBLK // SB

        def zrow(i, _):
            for h in range(fc // 16):
                zbuf[i, pl.ds(16 * h, 16)] = jnp.zeros((16,), jnp.float32)
            return 0

        lax.fori_loop(0, ZR, zrow, 0)

        for lk in range(2):
            chunk = cid * 2 + lk
            sbase = (chunk * TILES + sid) * NBLK
            dbase = sid * NBLK

            def zcp(r, _):
                pltpu.sync_copy(zbuf, acc.at[pl.ds(sid * rpt + r * ZR, ZR)])
                return 0

            lax.fori_loop(0, rpt // ZR, zcp, 0)
            plsc.subcore_barrier()

            def load_idx(s, p):
                return (pltpu.async_copy(
                            sidx_hbm.at[pl.ds(sbase + s * SB, SB)],
                            sidx_v.at[p], isem),
                        pltpu.async_copy(
                            didx_hbm.at[pl.ds(dbase + s * SB, SB)],
                            didx_v.at[p], isem))

            def pipeline(p):
                gh = [None] * SB
                sh = [None] * SB

                def scat(j):
                    gh[j].wait()
                    sh[j] = pltpu.async_copy(
                        rows_v.at[j % nbuf], acc.at[didx_v.at[p, j]], ssem,
                        add=True)

                for j in range(SB):
                    if j >= nbuf:
                        sh[j - nbuf].wait()
                    gh[j] = pltpu.async_copy(
                        src_hbm.at[sidx_v.at[p, j]], rows_v.at[j % nbuf],
                        gsem)
                    if j >= lead:
                        scat(j - lead)
                for j in range(SB - lead, SB):
                    scat(j)
                for j in range(SB - nbuf, SB):
                    sh[j].wait()

            for h in load_idx(0, 0):
                h.wait()

            def pair(t, _):
                h1 = load_idx(2 * t + 1, 1)
                pipeline(0)
                for h in h1:
                    h.wait()
                h0 = load_idx(2 * t + 2, 0)
                pipeline(1)
                for h in h0:
                    h.wait()
                return 0

            lax.fori_loop(0, (nsb - 1) // 2, pair, 0)
            pipeline(0)
            plsc.subcore_barrier()
            pltpu.sync_copy(
                acc.at[pl.ds(sid * rpt, rpt)],
                out_hbm.at[pl.ds(chunk * n_pad + sid * rpt, rpt)])
            plsc.subcore_barrier()

    return k


@functools.lru_cache(maxsize=None)
def _make_degree():
    n_pad = _pad128(2 * V)
    rpt = n_pad // TILES
    assert rpt % ZR == 0 and NBLK32 % SBD == 0

    @functools.partial(
        pl.kernel,
        mesh=_sc_mesh(),
        out_type=jax.ShapeDtypeStruct((2 * n_pad, 16), jnp.float32),
        scratch_types=[
            pltpu.VMEM((SBD, B), jnp.int32),
            pltpu.VMEM((B, 16), jnp.float32),
            pltpu.VMEM((ZR, 16), jnp.float32),
            pltpu.VMEM_SHARED((n_pad, 16), jnp.float32),
        ],
        compiler_params=pltpu.CompilerParams(use_tc_tiling_on_sc=False),
    )
    def k(didx_hbm, out_hbm, didx_v, ones_v, zbuf, acc):
        cid = lax.axis_index("c")
        sid = lax.axis_index("s")
        w = cid * TILES + sid

        def fill(i, _):
            zbuf[i] = jnp.zeros((16,), jnp.float32)
            return 0

        lax.fori_loop(0, ZR, fill, 0)

        def fill1(i, _):
            ones_v[i] = jnp.ones((16,), jnp.float32)
            return 0

        lax.fori_loop(0, B, fill1, 0)

        def zcp(r, _):
            pltpu.sync_copy(zbuf, acc.at[pl.ds(sid * rpt + r * ZR, ZR)])
            return 0

        lax.fori_loop(0, rpt // ZR, zcp, 0)
        plsc.subcore_barrier()

        def sblk(s, _):
            pltpu.sync_copy(didx_hbm.at[pl.ds(w * NBLK32 + s * SBD, SBD)],
                            didx_v)
            for j in range(SBD):
                pltpu.sync_copy(ones_v, acc.at[didx_v.at[j]], add=True)
            return 0

        lax.fori_loop(0, NBLK32 // SBD, sblk, 0)
        plsc.subcore_barrier()
        pltpu.sync_copy(acc.at[pl.ds(sid * rpt, rpt)],
                        out_hbm.at[pl.ds(cid * n_pad + sid * rpt, rpt)])

    return k


def _wb(p):
    return p[0], p[1].reshape(1, -1)


def _pn_fin(x, cs, old, count):
    xx = x - cs * (1.0 / (float(count) * float(count)))
    var = jnp.mean(xx * xx, axis=-1, keepdims=True)
    return xx * lax.rsqrt(var + EPS) * 0.25 + 0.1 * old


def _query_call(variables, noise, wq):
    (w1, b1), (w2, b2) = wq

    def body(v_ref, n_ref, w1r, b1r, w2r, b2r, q_ref, litsp_ref):
        v1 = jnp.concatenate([v_ref[...], n_ref[...]], axis=-1)
        h = _lrelu(jnp.dot(v1, w1r[...], preferred_element_type=jnp.float32)
                   + b1r[...])
        q = jnp.dot(h, w2r[...], preferred_element_type=jnp.float32) + b2r[...]
        q_ref[...] = q
        spq = _softplus(q)
        spn = _softplus(-q)
        for kk in range(4):
            litsp_ref[kk, 0] = spq[:, 16 * kk:16 * (kk + 1)]
            litsp_ref[kk, 1] = spn[:, 16 * kk:16 * (kk + 1)]

    return pl.pallas_call(
        body,
        grid=(NB_V,),
        in_specs=[
            pl.BlockSpec((RB_V, F), lambda i: (i, 0)),
            pl.BlockSpec((RB_V, 4), lambda i: (i, 0)),
            pl.BlockSpec((F + 4, F), lambda i: (0, 0)),
            pl.BlockSpec((1, F), lambda i: (0, 0)),
            pl.BlockSpec((F, F), lambda i: (0, 0)),
            pl.BlockSpec((1, F), lambda i: (0, 0)),
        ],
        out_specs=[
            pl.BlockSpec((RB_V, F), lambda i: (i, 0)),
            pl.BlockSpec((4, 2, RB_V, 16), lambda i: (0, 0, i, 0)),
        ],
        out_shape=[
            jax.ShapeDtypeStruct((V, F), jnp.float32),
            jax.ShapeDtypeStruct((4, 2, V, 16), jnp.float32),
        ],
    )(variables, noise, w1, b1, w2, b2)


def _query_fused_call(nv, csv, var_prev, noise, wq):
    (w1, b1), (w2, b2) = wq

    def body(nv_ref, cs_ref, old_ref, n_ref, w1r, b1r, w2r, b2r,
             v_ref, q_ref, litsp_ref):
        v = _pn_fin(nv_ref[...], cs_ref[...], old_ref[...], V)
        v_ref[...] = v
        v1 = jnp.concatenate([v, n_ref[...]], axis=-1)
        h = _lrelu(jnp.dot(v1, w1r[...], preferred_element_type=jnp.float32)
                   + b1r[...])
        q = jnp.dot(h, w2r[...], preferred_element_type=jnp.float32) + b2r[...]
        q_ref[...] = q
        spq = _softplus(q)
        spn = _softplus(-q)
        for kk in range(4):
            litsp_ref[kk, 0] = spq[:, 16 * kk:16 * (kk + 1)]
            litsp_ref[kk, 1] = spn[:, 16 * kk:16 * (kk + 1)]

    return pl.pallas_call(
        body,
        grid=(NB_V,),
        in_specs=[
            pl.BlockSpec((RB_V, F), lambda i: (i, 0)),
            pl.BlockSpec((1, F), lambda i: (0, 0)),
            pl.BlockSpec((RB_V, F), lambda i: (i, 0)),
            pl.BlockSpec((RB_V, 4), lambda i: (i, 0)),
            pl.BlockSpec((F + 4, F), lambda i: (0, 0)),
            pl.BlockSpec((1, F), lambda i: (0, 0)),
            pl.BlockSpec((F, F), lambda i: (0, 0)),
            pl.BlockSpec((1, F), lambda i: (0, 0)),
        ],
        out_specs=[
            pl.BlockSpec((RB_V, F), lambda i: (i, 0)),
            pl.BlockSpec((RB_V, F), lambda i: (i, 0)),
            pl.BlockSpec((4, 2, RB_V, 16), lambda i: (0, 0, i, 0)),
        ],
        out_shape=[
            jax.ShapeDtypeStruct((V, F), jnp.float32),
            jax.ShapeDtypeStruct((V, F), jnp.float32),
            jax.ShapeDtypeStruct((4, 2, V, 16), jnp.float32),
        ],
    )(nv, csv, var_prev, noise, w1, b1, w2, b2)


def _clause_fused_call(cv4, ncv_p, cs_p, old, wc):
    (w1, b1), (w2, b2) = wc

    def body(cv_ref, np_ref, cp_ref, old_ref, w1r, b1r, w2r, b2r,
             cf_ref, src_ref, ncv_ref, cs_ref):
        cls = _pn_fin(np_ref[...], cp_ref[...], old_ref[...], C)
        cf_ref[...] = cls
        cv = jnp.concatenate([cv_ref[kk] for kk in range(4)], axis=-1)
        closs = jnp.exp(-cv)
        cu = jnp.concatenate([cls, 4.0 * closs], axis=-1)
        h = _lrelu(jnp.dot(cu, w1r[...], preferred_element_type=jnp.float32)
                   + b1r[...])
        cd = jnp.dot(h, w2r[...], preferred_element_type=jnp.float32) + b2r[...]
        vla = cd[:, :F]
        ncv = cd[:, F:]
        ncs = -closs
        src_ref[0] = ncs[:, :32]
        src_ref[1] = ncs[:, 32:]
        src_ref[2] = vla[:, :32]
        src_ref[3] = vla[:, 32:]
        ncv_ref[...] = ncv

        @pl.when(pl.program_id(0) == 0)
        def _():
            cs_ref[...] = jnp.zeros_like(cs_ref)

        cs_ref[...] += jnp.sum(ncv, axis=0, keepdims=True)

    return pl.pallas_call(
        body,
        grid=(NB_C,),
        in_specs=[
            pl.BlockSpec((4, RB_C, 16), lambda i: (0, i, 0)),
            pl.BlockSpec((RB_C, F), lambda i: (i, 0)),
            pl.BlockSpec((1, F), lambda i: (0, 0)),
            pl.BlockSpec((RB_C, F), lambda i: (i, 0)),
            pl.BlockSpec((2 * F, 2 * F), lambda i: (0, 0)),
            pl.BlockSpec((1, 2 * F), lambda i: (0, 0)),
            pl.BlockSpec((2 * F, 2 * F), lambda i: (0, 0)),
            pl.BlockSpec((1, 2 * F), lambda i: (0, 0)),
        ],
        out_specs=[
            pl.BlockSpec((RB_C, F), lambda i: (i, 0)),
            pl.BlockSpec((4, RB_C, 32), lambda i: (0, i, 0)),
            pl.BlockSpec((RB_C, F), lambda i: (i, 0)),
            pl.BlockSpec((1, F), lambda i: (0, 0)),
        ],
        out_shape=[
            jax.ShapeDtypeStruct((C, F), jnp.float32),
            jax.ShapeDtypeStruct((4, C, 32), jnp.float32),
            jax.ShapeDtypeStruct((C, F), jnp.float32),
            jax.ShapeDtypeStruct((1, F), jnp.float32),
        ],
    )(cv4, ncv_p, cs_p, old, w1, b1, w2, b2)


def _out_fused_call(ncv_p, cs_p, old, wo):
    (w1, b1), (w2, b2) = wo

    def body(np_ref, cp_ref, old_ref, w1r, b1r, w2r, b2r, sig_ref, sp_ref):
        cls = _pn_fin(np_ref[...], cp_ref[...], old_ref[...], C)
        h = _lrelu(jnp.dot(cls, w1r[...],
                           preferred_element_type=jnp.float32) + b1r[...])
        logit = jnp.dot(h, w2r[...], preferred_element_type=jnp.float32) + b2r[...]
        sig_ref[...] = _sigmoid(logit)
        sp_ref[...] = _softplus(logit)

    return pl.pallas_call(
        body,
        grid=(NB_C,),
        in_specs=[
            pl.BlockSpec((RB_C, F), lambda i: (i, 0)),
            pl.BlockSpec((1, F), lambda i: (0, 0)),
            pl.BlockSpec((RB_C, F), lambda i: (i, 0)),
            pl.BlockSpec((F, F), lambda i: (0, 0)),
            pl.BlockSpec((1, F), lambda i: (0, 0)),
            pl.BlockSpec((F, 1), lambda i: (0, 0)),
            pl.BlockSpec((1, 1), lambda i: (0, 0)),
        ],
        out_specs=[
            pl.BlockSpec((RB_C, 1), lambda i: (i, 0)),
            pl.BlockSpec((RB_C, 1), lambda i: (i, 0)),
        ],
        out_shape=[
            jax.ShapeDtypeStruct((C, 1), jnp.float32),
            jax.ShapeDtypeStruct((C, 1), jnp.float32),
        ],
    )(ncv_p, cs_p, old, w1, b1, w2, b2)


def _clause_call(cv4, clauses, wc):
    (w1, b1), (w2, b2) = wc

    def body(cv_ref, c_ref, w1r, b1r, w2r, b2r, src_ref, ncv_ref, cs_ref):
        cv = jnp.concatenate([cv_ref[kk] for kk in range(4)], axis=-1)
        closs = jnp.exp(-cv)
        cu = jnp.concatenate([c_ref[...], 4.0 * closs], axis=-1)
        h = _lrelu(jnp.dot(cu, w1r[...], preferred_element_type=jnp.float32)
                   + b1r[...])
        cd = jnp.dot(h, w2r[...], preferred_element_type=jnp.float32) + b2r[...]
        vla = cd[:, :F]
        ncv = cd[:, F:]
        ncs = -closs
        src_ref[0] = ncs[:, :32]
        src_ref[1] = ncs[:, 32:]
        src_ref[2] = vla[:, :32]
        src_ref[3] = vla[:, 32:]
        ncv_ref[...] = ncv

        @pl.when(pl.program_id(0) == 0)
        def _():
            cs_ref[...] = jnp.zeros_like(cs_ref)

        cs_ref[...] += jnp.sum(ncv, axis=0, keepdims=True)

    return pl.pallas_call(
        body,
        grid=(NB_C,),
        in_specs=[
            pl.BlockSpec((4, RB_C, 16), lambda i: (0, i, 0)),
            pl.BlockSpec((RB_C, F), lambda i: (i, 0)),
            pl.BlockSpec((2 * F, 2 * F), lambda i: (0, 0)),
            pl.BlockSpec((1, 2 * F), lambda i: (0, 0)),
            pl.BlockSpec((2 * F, 2 * F), lambda i: (0, 0)),
            pl.BlockSpec((1, 2 * F), lambda i: (0, 0)),
        ],
        out_specs=[
            pl.BlockSpec((4, RB_C, 32), lambda i: (0, i, 0)),
            pl.BlockSpec((RB_C, F), lambda i: (i, 0)),
            pl.BlockSpec((1, F), lambda i: (0, 0)),
        ],
        out_shape=[
            jax.ShapeDtypeStruct((4, C, 32), jnp.float32),
            jax.ShapeDtypeStruct((C, F), jnp.float32),
            jax.ShapeDtypeStruct((1, F), jnp.float32),
        ],
    )(cv4, clauses, w1, b1, w2, b2)


def _update_call(q, acc4, variables, deg4, wg):
    (w1, b1), (w2, b2), (w3, b3) = wg

    def body(q_ref, a_ref, v_ref, d_ref, w1r, b1r, w2r, b2r, w3r, b3r,
             nv_ref, cs_ref):
        q = q_ref[...]
        t_pos = jnp.concatenate([a_ref[0, 0], a_ref[1, 0]], axis=-1)
        t_neg = jnp.concatenate([a_ref[0, 1], a_ref[1, 1]], axis=-1)
        vlp = jnp.concatenate([a_ref[2, 0], a_ref[3, 0]], axis=-1)
        vln = jnp.concatenate([a_ref[2, 1], a_ref[3, 1]], axis=-1)
        deg = d_ref[0] + d_ref[1]
        degp = deg[0][:, 0:1]
        degn = deg[1][:, 0:1]
        dwp = lax.rsqrt(jnp.maximum(degp, 1.0))
        dwn = lax.rsqrt(jnp.maximum(degn, 1.0))
        vdw = 4.0 * lax.rsqrt(jnp.maximum(degp + degn, 1.0))
        qg = _sigmoid(q) * t_pos - _sigmoid(-q) * t_neg
        unit = jnp.concatenate(
            [qg * vdw, v_ref[...], vlp * dwp, vln * dwn], axis=-1)
        h1 = _lrelu(jnp.dot(unit, w1r[...], preferred_element_type=jnp.float32)
                    + b1r[...])
        h2 = _lrelu(jnp.dot(h1, w2r[...], preferred_element_type=jnp.float32)
                    + b2r[...])
        nv = jnp.dot(h2, w3r[...], preferred_element_type=jnp.float32) + b3r[...]
        nv_ref[...] = nv

        @pl.when(pl.program_id(0) == 0)
        def _():
            cs_ref[...] = jnp.zeros_like(cs_ref)

        cs_ref[...] += jnp.sum(nv, axis=0, keepdims=True)

    return pl.pallas_call(
        body,
        grid=(NB_V,),
        in_specs=[
            pl.BlockSpec((RB_V, F), lambda i: (i, 0)),
            pl.BlockSpec((4, 2, RB_V, 32), lambda i: (0, 0, i, 0)),
            pl.BlockSpec((RB_V, F), lambda i: (i, 0)),
            pl.BlockSpec((2, 2, RB_V, 16), lambda i: (0, 0, i, 0)),
            pl.BlockSpec((4 * F, 2 * F), lambda i: (0, 0)),
            pl.BlockSpec((1, 2 * F), lambda i: (0, 0)),
            pl.BlockSpec((2 * F, 2 * F), lambda i: (0, 0)),
            pl.BlockSpec((1, 2 * F), lambda i: (0, 0)),
            pl.BlockSpec((2 * F, F), lambda i: (0, 0)),
            pl.BlockSpec((1, F), lambda i: (0, 0)),
        ],
        out_specs=[
            pl.BlockSpec((RB_V, F), lambda i: (i, 0)),
            pl.BlockSpec((1, F), lambda i: (0, 0)),
        ],
        out_shape=[
            jax.ShapeDtypeStruct((V, F), jnp.float32),
            jax.ShapeDtypeStruct((1, F), jnp.float32),
        ],
    )(q, acc4, variables, deg4, w1, b1, w2, b2, w3, b3)


def kernel(params, lit_idx, clause_idx, clause_graph_ids, var_graph_ids):
    wq = tuple(_wb(p) for p in params['variables_query'])
    wc = tuple(_wb(p) for p in params['clause_mlp'])
    wg = tuple(_wb(p) for p in params['update_gate'])
    wo = tuple(_wb(p) for p in params['clauses_output'])

    noise = jnp.stack([
        jax.random.normal(jax.random.fold_in(jax.random.key(42), s), (V, 4),
                          dtype=jnp.float32) for s in range(ROUNDS)])

    off_a = (jnp.arange(4, dtype=jnp.int32) * (2 * V))[:, None]
    sidx_a = (lit_idx[None, :] + off_a).reshape(4 * TILES * NBLK, B)
    didx_a = clause_idx.reshape(TILES * NBLK, B)
    off_b = (jnp.arange(4, dtype=jnp.int32) * C)[:, None]
    sidx_b = (clause_idx[None, :] + off_b).reshape(4 * TILES * NBLK, B)
    hoff = _pad128(2 * V) // 2 - V
    didx_b = jnp.where(lit_idx < V, lit_idx,
                       lit_idx + hoff).reshape(TILES * NBLK, B)
    didx_d = lit_idx.reshape(32 * NBLK32, B)

    pad_l = _pad128(2 * V)
    pad_c = _pad128(C)
    deg4 = (_make_degree()(didx_d)
            .reshape(2, pad_l, 16)[:, :2 * V]
            .reshape(2, 2, V, 16))

    seg_a = _make_seg_sum(C, 16)
    seg_b = _make_seg_sum(2 * V, 32)

    ones_v = jnp.ones((V, F), jnp.float32)
    ones_c = jnp.ones((C, F), jnp.float32)

    variables = ones_v
    var_old = ones_v
    cls_old = ones_c
    nv = cs_v = ncv = cs_c = None

    for step in range(ROUNDS):
        if step == 0:
            q, litsp = _query_call(variables, noise[step], wq)
        else:
            variables, q, litsp = _query_fused_call(
                nv, cs_v, var_old, noise[step], wq)
        cv4 = seg_a(litsp.reshape(8 * V, 16), sidx_a, didx_a).reshape(
            4, pad_c, 16)
        if step == 0:
            src4, ncv, cs_c = _clause_call(cv4, cls_old, wc)
        else:
            cls_old, src4, ncv, cs_c = _clause_fused_call(
                cv4, ncv, cs_c, cls_old, wc)
        acc4 = seg_b(src4.reshape(4 * C, 32), sidx_b, didx_b).reshape(
            4, 2, pad_l // 2, 32)
        var_old = variables
        nv, cs_v = _update_call(q, acc4, variables, deg4, wg)

    sig, sp = _out_fused_call(ncv, cs_c, cls_old, wo)
    return (sig[:, 0], sp[:, 0])
